# baseline (speedup 1.0000x reference)
import functools
import jax
import jax.numpy as jnp
from jax import lax
from jax.experimental import pallas as pl
from jax.experimental.pallas import tpu as pltpu
from jax.experimental.pallas import tpu_sc as plsc

N = 10000
D = 128
NC, NS, LANES = 2, 16, 16
NW = NC * NS
CHUNK = 128
EDGE_CHUNKS = 80
IDX_BLK = 16
E_PAD = NW * EDGE_CHUNKS * CHUNK
TOT_CHUNKS = E_PAD // CHUNK
C0_CHUNKS = 160
C1_CHUNKS = 0
assert C0_CHUNKS + C1_CHUNKS == 2 * EDGE_CHUNKS
assert C0_CHUNKS % IDX_BLK == 0 and C1_CHUNKS % IDX_BLK == 0
ACC_ROWS = 10240
ROWS_PER_TILE = ACC_ROWS // NS
WB_CHUNKS = ROWS_PER_TILE // CHUNK
DUMMY_DST = N

_MESH = plsc.VectorSubcoreMesh(core_axis_name="c", subcore_axis_name="s")


def _zero_buf(buf, rows, width):
    def zrow(i, carry):
        def zcol(k, c2):
            buf[i, pl.ds(k * LANES, LANES)] = jnp.zeros((LANES,), jnp.float32)
            return c2
        return lax.fori_loop(0, width // LANES, zcol, carry)
    lax.fori_loop(0, rows, zrow, 0)


@functools.partial(
    pl.kernel,
    out_type=jax.ShapeDtypeStruct((NC, ACC_ROWS, LANES), jnp.float32),
    mesh=_MESH,
    scratch_types=[
        pltpu.VMEM((EDGE_CHUNKS, CHUNK), jnp.int32),
        pltpu.VMEM((CHUNK, LANES), jnp.float32),
        pltpu.VMEM_SHARED((ACC_ROWS, LANES), jnp.float32),
    ],
)
def _deg_kernel(dst_hbm, out_hbm, dst_v, ones_v, acc):
    c = lax.axis_index("c")
    s = lax.axis_index("s")
    wid = c * NS + s
    base = s * ROWS_PER_TILE

    _zero_buf(ones_v, CHUNK, LANES)
    for k in range(WB_CHUNKS):
        pltpu.sync_copy(ones_v, acc.at[pl.ds(base + k * CHUNK, CHUNK)])
    plsc.subcore_barrier()

    def orow(i, carry):
        ones_v[i, pl.ds(0, LANES)] = jnp.ones((LANES,), jnp.float32)
        return carry
    lax.fori_loop(0, CHUNK, orow, 0)

    pltpu.sync_copy(dst_hbm.at[pl.ds(wid * EDGE_CHUNKS, EDGE_CHUNKS)], dst_v)

    def body(j, carry):
        pltpu.sync_copy(ones_v, acc.at[dst_v.at[j]], add=True)
        return carry
    lax.fori_loop(0, EDGE_CHUNKS, body, 0)

    plsc.subcore_barrier()
    for k in range(WB_CHUNKS):
        pltpu.sync_copy(acc.at[pl.ds(base + k * CHUNK, CHUNK)],
                        out_hbm.at[c, pl.ds(base + k * CHUNK, CHUNK)])


@functools.partial(
    pl.kernel,
    out_type=jax.ShapeDtypeStruct((NC, ACC_ROWS, D), jnp.float32),
    mesh=_MESH,
    scratch_types=[
        pltpu.VMEM((IDX_BLK, CHUNK), jnp.int32),
        pltpu.VMEM((IDX_BLK, CHUNK), jnp.int32),
        pltpu.VMEM((CHUNK, D), jnp.float32),
        pltpu.VMEM((CHUNK, D), jnp.float32),
        pltpu.VMEM_SHARED((ACC_ROWS, D), jnp.float32),
        pltpu.SemaphoreType.DMA,
        pltpu.SemaphoreType.DMA,
    ],
)
def _edge_kernel(src_hbm, dst_hbm, hp_hbm, out_hbm, src_v, dst_v, buf0, buf1,
                 acc, sem0, sem1):
    c = lax.axis_index("c")
    s = lax.axis_index("s")
    wid = c * NS + s
    base = s * ROWS_PER_TILE

    _zero_buf(buf0, CHUNK, D)
    for k in range(WB_CHUNKS):
        pltpu.sync_copy(buf0, acc.at[pl.ds(base + k * CHUNK, CHUNK)])
    plsc.subcore_barrier()

    start_chunk = jnp.where(c == 0, s * C0_CHUNKS, NS * C0_CHUNKS + s * C1_CHUNKS)
    nblocks = jnp.where(c == 0, C0_CHUNKS // IDX_BLK, C1_CHUNKS // IDX_BLK)

    def block(b, carry):
        off = start_chunk + b * IDX_BLK
        pltpu.sync_copy(src_hbm.at[pl.ds(off, IDX_BLK)], src_v)
        pltpu.sync_copy(dst_hbm.at[pl.ds(off, IDX_BLK)], dst_v)
        pltpu.async_copy(hp_hbm.at[src_v.at[0]], buf0, sem0)

        def body(t, c2):
            j0 = 2 * t
            j1 = j0 + 1
            pltpu.async_copy(hp_hbm.at[src_v.at[j1]], buf1, sem1)
            pltpu.make_async_copy(hp_hbm.at[src_v.at[j0]], buf0, sem0).wait()
            pltpu.sync_copy(buf0, acc.at[dst_v.at[j0]], add=True)
            jn = jnp.minimum(j0 + 2, IDX_BLK - 2)
            pltpu.async_copy(hp_hbm.at[src_v.at[jn]], buf0, sem0)
            pltpu.make_async_copy(hp_hbm.at[src_v.at[j1]], buf1, sem1).wait()
            pltpu.sync_copy(buf1, acc.at[dst_v.at[j1]], add=True)
            return c2
        lax.fori_loop(0, IDX_BLK // 2, body, 0)
        pltpu.make_async_copy(hp_hbm.at[src_v.at[IDX_BLK - 2]], buf0, sem0).wait()
        return carry
    lax.fori_loop(0, nblocks, block, 0)

    plsc.subcore_barrier()
    for k in range(WB_CHUNKS):
        pltpu.sync_copy(acc.at[pl.ds(base + k * CHUNK, CHUNK)],
                        out_hbm.at[c, pl.ds(base + k * CHUNK, CHUNK)])


BN = 1000


def _dinv_of(deg_ref):
    d = deg_ref[0] + deg_ref[1]
    return lax.rsqrt(1.0 + d[:, 0:1])


def _mm1_body(deg_ref, x_ref, w_ref, hp_ref):
    dinv = _dinv_of(deg_ref)
    h = jnp.dot(x_ref[...], w_ref[...], preferred_element_type=jnp.float32)
    hp_ref[...] = dinv * h


def _mid_body(deg_ref, agg_ref, hp1_ref, b1_ref, w2_ref, hp2_ref):
    dinv = _dinv_of(deg_ref)
    aggsum = agg_ref[0] + agg_ref[1]
    z = jnp.maximum(dinv * (aggsum + hp1_ref[...]) + b1_ref[...], 0.0)
    hp2_ref[...] = dinv * jnp.dot(z, w2_ref[...], preferred_element_type=jnp.float32)


def _out_body(deg_ref, agg_ref, hp2_ref, b2_ref, o_ref):
    dinv = _dinv_of(deg_ref)
    aggsum = agg_ref[0] + agg_ref[1]
    o_ref[...] = dinv * (aggsum + hp2_ref[...]) + b2_ref[...]


_deg_spec = pl.BlockSpec((NC, BN, LANES), lambda i: (0, i, 0))
_agg_spec = pl.BlockSpec((NC, BN, D), lambda i: (0, i, 0))
_row_spec = pl.BlockSpec((BN, D), lambda i: (i, 0))
_w_spec = pl.BlockSpec((D, D), lambda i: (0, 0))
_b_spec = pl.BlockSpec((1, D), lambda i: (0, 0))

_mm1 = pl.pallas_call(
    _mm1_body,
    grid=(N // BN,),
    in_specs=[_deg_spec, _row_spec, _w_spec],
    out_specs=_row_spec,
    out_shape=jax.ShapeDtypeStruct((N, D), jnp.float32),
)

_mid = pl.pallas_call(
    _mid_body,
    grid=(N // BN,),
    in_specs=[_deg_spec, _agg_spec, _row_spec, _b_spec, _w_spec],
    out_specs=_row_spec,
    out_shape=jax.ShapeDtypeStruct((N, D), jnp.float32),
)

_outk = pl.pallas_call(
    _out_body,
    grid=(N // BN,),
    in_specs=[_deg_spec, _agg_spec, _row_spec, _b_spec],
    out_specs=_row_spec,
    out_shape=jax.ShapeDtypeStruct((N, D), jnp.float32),
)


def kernel(x, edge_index, W1, b1, W2, b2):
    e = edge_index.shape[1]
    pad = E_PAD - e
    src = jnp.concatenate([edge_index[0], jnp.zeros((pad,), jnp.int32)])
    dst = jnp.concatenate([edge_index[1], jnp.full((pad,), DUMMY_DST, jnp.int32)])
    src_p = src.reshape(TOT_CHUNKS, CHUNK)
    dst_p = dst.reshape(TOT_CHUNKS, CHUNK)

    deg_parts = _deg_kernel(dst_p)
    hp1 = _mm1(deg_parts, x, W1)
    agg1 = _edge_kernel(src_p, dst_p, hp1)
    hp2 = _mid(deg_parts, agg1, hp1, b1.reshape(1, D), W2)
    agg2 = _edge_kernel(src_p, dst_p, hp2)
    return _outk(deg_parts, agg2, hp2, b2.reshape(1, D))


# speedup vs baseline: 1.2593x; 1.2593x over previous
import functools
import jax
import jax.numpy as jnp
from jax import lax
from jax.experimental import pallas as pl
from jax.experimental.pallas import tpu as pltpu
from jax.experimental.pallas import tpu_sc as plsc

N = 10000
D = 128
NC, NS, LANES = 2, 16, 16
NW = NC * NS
CHUNK = 128
EDGE_CHUNKS = 80
IDX_BLK = 16
E_PAD = NW * EDGE_CHUNKS * CHUNK
TOT_CHUNKS = E_PAD // CHUNK
C0_CHUNKS = 112
C1_CHUNKS = 48
assert C0_CHUNKS + C1_CHUNKS == 2 * EDGE_CHUNKS
assert C0_CHUNKS % IDX_BLK == 0 and C1_CHUNKS % IDX_BLK == 0
ACC_ROWS = 10240
ROWS_PER_TILE = ACC_ROWS // NS
WB_CHUNKS = ROWS_PER_TILE // CHUNK
DUMMY_DST = N

_MESH = plsc.VectorSubcoreMesh(core_axis_name="c", subcore_axis_name="s")


def _zero_buf(buf, rows, width):
    def zrow(i, carry):
        def zcol(k, c2):
            buf[i, pl.ds(k * LANES, LANES)] = jnp.zeros((LANES,), jnp.float32)
            return c2
        return lax.fori_loop(0, width // LANES, zcol, carry)
    lax.fori_loop(0, rows, zrow, 0)


@functools.partial(
    pl.kernel,
    out_type=jax.ShapeDtypeStruct((NC, ACC_ROWS, LANES), jnp.float32),
    mesh=_MESH,
    scratch_types=[
        pltpu.VMEM((EDGE_CHUNKS, CHUNK), jnp.int32),
        pltpu.VMEM((CHUNK, LANES), jnp.float32),
        pltpu.VMEM_SHARED((ACC_ROWS, LANES), jnp.float32),
    ],
)
def _deg_kernel(dst_hbm, out_hbm, dst_v, ones_v, acc):
    c = lax.axis_index("c")
    s = lax.axis_index("s")
    wid = c * NS + s
    base = s * ROWS_PER_TILE

    _zero_buf(ones_v, CHUNK, LANES)
    for k in range(WB_CHUNKS):
        pltpu.sync_copy(ones_v, acc.at[pl.ds(base + k * CHUNK, CHUNK)])
    plsc.subcore_barrier()

    def orow(i, carry):
        ones_v[i, pl.ds(0, LANES)] = jnp.ones((LANES,), jnp.float32)
        return carry
    lax.fori_loop(0, CHUNK, orow, 0)

    pltpu.sync_copy(dst_hbm.at[pl.ds(wid * EDGE_CHUNKS, EDGE_CHUNKS)], dst_v)

    def body(j, carry):
        pltpu.sync_copy(ones_v, acc.at[dst_v.at[j]], add=True)
        return carry
    lax.fori_loop(0, EDGE_CHUNKS, body, 0)

    plsc.subcore_barrier()
    for k in range(WB_CHUNKS):
        pltpu.sync_copy(acc.at[pl.ds(base + k * CHUNK, CHUNK)],
                        out_hbm.at[c, pl.ds(base + k * CHUNK, CHUNK)])


@functools.partial(
    pl.kernel,
    out_type=jax.ShapeDtypeStruct((NC, ACC_ROWS, D), jnp.float32),
    mesh=_MESH,
    scratch_types=[
        pltpu.VMEM((IDX_BLK, CHUNK), jnp.int32),
        pltpu.VMEM((IDX_BLK, CHUNK), jnp.int32),
        pltpu.VMEM((CHUNK, D), jnp.float32),
        pltpu.VMEM((CHUNK, D), jnp.float32),
        pltpu.VMEM_SHARED((ACC_ROWS, D), jnp.float32),
        pltpu.SemaphoreType.DMA,
        pltpu.SemaphoreType.DMA,
    ],
)
def _edge_kernel(src_hbm, dst_hbm, hp_hbm, out_hbm, src_v, dst_v, buf0, buf1,
                 acc, sem0, sem1):
    c = lax.axis_index("c")
    s = lax.axis_index("s")
    wid = c * NS + s
    base = s * ROWS_PER_TILE

    _zero_buf(buf0, CHUNK, D)
    for k in range(WB_CHUNKS):
        pltpu.sync_copy(buf0, acc.at[pl.ds(base + k * CHUNK, CHUNK)])
    plsc.subcore_barrier()

    start_chunk = jnp.where(c == 0, s * C0_CHUNKS, NS * C0_CHUNKS + s * C1_CHUNKS)
    nblocks = jnp.where(c == 0, C0_CHUNKS // IDX_BLK, C1_CHUNKS // IDX_BLK)

    def block(b, carry):
        off = start_chunk + b * IDX_BLK
        pltpu.sync_copy(src_hbm.at[pl.ds(off, IDX_BLK)], src_v)
        pltpu.sync_copy(dst_hbm.at[pl.ds(off, IDX_BLK)], dst_v)
        pltpu.async_copy(hp_hbm.at[src_v.at[0]], buf0, sem0)

        def body(t, c2):
            j0 = 2 * t
            j1 = j0 + 1
            pltpu.async_copy(hp_hbm.at[src_v.at[j1]], buf1, sem1)
            pltpu.make_async_copy(hp_hbm.at[src_v.at[j0]], buf0, sem0).wait()
            pltpu.sync_copy(buf0, acc.at[dst_v.at[j0]], add=True)
            jn = jnp.minimum(j0 + 2, IDX_BLK - 2)
            pltpu.async_copy(hp_hbm.at[src_v.at[jn]], buf0, sem0)
            pltpu.make_async_copy(hp_hbm.at[src_v.at[j1]], buf1, sem1).wait()
            pltpu.sync_copy(buf1, acc.at[dst_v.at[j1]], add=True)
            return c2
        lax.fori_loop(0, IDX_BLK // 2, body, 0)
        pltpu.make_async_copy(hp_hbm.at[src_v.at[IDX_BLK - 2]], buf0, sem0).wait()
        return carry
    lax.fori_loop(0, nblocks, block, 0)

    plsc.subcore_barrier()
    for k in range(WB_CHUNKS):
        pltpu.sync_copy(acc.at[pl.ds(base + k * CHUNK, CHUNK)],
                        out_hbm.at[c, pl.ds(base + k * CHUNK, CHUNK)])


BN = 1000


def _dinv_of(deg_ref):
    d = deg_ref[0] + deg_ref[1]
    return lax.rsqrt(1.0 + d[:, 0:1])


def _mm1_body(deg_ref, x_ref, w_ref, hp_ref):
    dinv = _dinv_of(deg_ref)
    h = jnp.dot(x_ref[...], w_ref[...], preferred_element_type=jnp.float32)
    hp_ref[...] = dinv * h


def _mid_body(deg_ref, agg_ref, hp1_ref, b1_ref, w2_ref, hp2_ref):
    dinv = _dinv_of(deg_ref)
    aggsum = agg_ref[0] + agg_ref[1]
    z = jnp.maximum(dinv * (aggsum + hp1_ref[...]) + b1_ref[...], 0.0)
    hp2_ref[...] = dinv * jnp.dot(z, w2_ref[...], preferred_element_type=jnp.float32)


def _out_body(deg_ref, agg_ref, hp2_ref, b2_ref, o_ref):
    dinv = _dinv_of(deg_ref)
    aggsum = agg_ref[0] + agg_ref[1]
    o_ref[...] = dinv * (aggsum + hp2_ref[...]) + b2_ref[...]


_deg_spec = pl.BlockSpec((NC, BN, LANES), lambda i: (0, i, 0))
_agg_spec = pl.BlockSpec((NC, BN, D), lambda i: (0, i, 0))
_row_spec = pl.BlockSpec((BN, D), lambda i: (i, 0))
_w_spec = pl.BlockSpec((D, D), lambda i: (0, 0))
_b_spec = pl.BlockSpec((1, D), lambda i: (0, 0))

_mm1 = pl.pallas_call(
    _mm1_body,
    grid=(N // BN,),
    in_specs=[_deg_spec, _row_spec, _w_spec],
    out_specs=_row_spec,
    out_shape=jax.ShapeDtypeStruct((N, D), jnp.float32),
)

_mid = pl.pallas_call(
    _mid_body,
    grid=(N // BN,),
    in_specs=[_deg_spec, _agg_spec, _row_spec, _b_spec, _w_spec],
    out_specs=_row_spec,
    out_shape=jax.ShapeDtypeStruct((N, D), jnp.float32),
)

_outk = pl.pallas_call(
    _out_body,
    grid=(N // BN,),
    in_specs=[_deg_spec, _agg_spec, _row_spec, _b_spec],
    out_specs=_row_spec,
    out_shape=jax.ShapeDtypeStruct((N, D), jnp.float32),
)


def kernel(x, edge_index, W1, b1, W2, b2):
    e = edge_index.shape[1]
    pad = E_PAD - e
    src = jnp.concatenate([edge_index[0], jnp.zeros((pad,), jnp.int32)])
    dst = jnp.concatenate([edge_index[1], jnp.full((pad,), DUMMY_DST, jnp.int32)])
    src_p = src.reshape(TOT_CHUNKS, CHUNK)
    dst_p = dst.reshape(TOT_CHUNKS, CHUNK)

    deg_parts = _deg_kernel(dst_p)
    hp1 = _mm1(deg_parts, x, W1)
    agg1 = _edge_kernel(src_p, dst_p, hp1)
    hp2 = _mid(deg_parts, agg1, hp1, b1.reshape(1, D), W2)
    agg2 = _edge_kernel(src_p, dst_p, hp2)
    return _outk(deg_parts, agg2, hp2, b2.reshape(1, D))


# speedup vs baseline: 1.4317x; 1.1369x over previous
import functools
import jax
import jax.numpy as jnp
from jax import lax
from jax.experimental import pallas as pl
from jax.experimental.pallas import tpu as pltpu
from jax.experimental.pallas import tpu_sc as plsc

N = 10000
D = 128
NC, NS, LANES = 2, 16, 16
NW = NC * NS
CHUNK = 128
EDGE_CHUNKS = 80
IDX_BLK = 16
E_PAD = NW * EDGE_CHUNKS * CHUNK
TOT_CHUNKS = E_PAD // CHUNK
C0_CHUNKS = 144
C1_CHUNKS = 16
assert C0_CHUNKS + C1_CHUNKS == 2 * EDGE_CHUNKS
assert C0_CHUNKS % IDX_BLK == 0 and C1_CHUNKS % IDX_BLK == 0
ACC_ROWS = 10240
ROWS_PER_TILE = ACC_ROWS // NS
WB_CHUNKS = ROWS_PER_TILE // CHUNK
DUMMY_DST = N

_MESH = plsc.VectorSubcoreMesh(core_axis_name="c", subcore_axis_name="s")


def _zero_buf(buf, rows, width):
    def zrow(i, carry):
        def zcol(k, c2):
            buf[i, pl.ds(k * LANES, LANES)] = jnp.zeros((LANES,), jnp.float32)
            return c2
        return lax.fori_loop(0, width // LANES, zcol, carry)
    lax.fori_loop(0, rows, zrow, 0)


@functools.partial(
    pl.kernel,
    out_type=jax.ShapeDtypeStruct((NC, ACC_ROWS, LANES), jnp.float32),
    mesh=_MESH,
    scratch_types=[
        pltpu.VMEM((EDGE_CHUNKS, CHUNK), jnp.int32),
        pltpu.VMEM((CHUNK, LANES), jnp.float32),
        pltpu.VMEM_SHARED((ACC_ROWS, LANES), jnp.float32),
    ],
)
def _deg_kernel(dst_hbm, out_hbm, dst_v, ones_v, acc):
    c = lax.axis_index("c")
    s = lax.axis_index("s")
    wid = c * NS + s
    base = s * ROWS_PER_TILE

    _zero_buf(ones_v, CHUNK, LANES)
    for k in range(WB_CHUNKS):
        pltpu.sync_copy(ones_v, acc.at[pl.ds(base + k * CHUNK, CHUNK)])
    plsc.subcore_barrier()

    def orow(i, carry):
        ones_v[i, pl.ds(0, LANES)] = jnp.ones((LANES,), jnp.float32)
        return carry
    lax.fori_loop(0, CHUNK, orow, 0)

    pltpu.sync_copy(dst_hbm.at[pl.ds(wid * EDGE_CHUNKS, EDGE_CHUNKS)], dst_v)

    def body(j, carry):
        pltpu.sync_copy(ones_v, acc.at[dst_v.at[j]], add=True)
        return carry
    lax.fori_loop(0, EDGE_CHUNKS, body, 0)

    plsc.subcore_barrier()
    for k in range(WB_CHUNKS):
        pltpu.sync_copy(acc.at[pl.ds(base + k * CHUNK, CHUNK)],
                        out_hbm.at[c, pl.ds(base + k * CHUNK, CHUNK)])


@functools.partial(
    pl.kernel,
    out_type=jax.ShapeDtypeStruct((NC, ACC_ROWS, D), jnp.float32),
    mesh=_MESH,
    scratch_types=[
        pltpu.VMEM((IDX_BLK, CHUNK), jnp.int32),
        pltpu.VMEM((IDX_BLK, CHUNK), jnp.int32),
        pltpu.VMEM((CHUNK, D), jnp.float32),
        pltpu.VMEM((CHUNK, D), jnp.float32),
        pltpu.VMEM_SHARED((ACC_ROWS, D), jnp.float32),
        pltpu.SemaphoreType.DMA,
        pltpu.SemaphoreType.DMA,
    ],
)
def _edge_kernel(src_hbm, dst_hbm, hp_hbm, out_hbm, src_v, dst_v, buf0, buf1,
                 acc, sem0, sem1):
    c = lax.axis_index("c")
    s = lax.axis_index("s")
    wid = c * NS + s
    base = s * ROWS_PER_TILE

    _zero_buf(buf0, CHUNK, D)
    for k in range(WB_CHUNKS):
        pltpu.sync_copy(buf0, acc.at[pl.ds(base + k * CHUNK, CHUNK)])
    plsc.subcore_barrier()

    start_chunk = jnp.where(c == 0, s * C0_CHUNKS, NS * C0_CHUNKS + s * C1_CHUNKS)
    nblocks = jnp.where(c == 0, C0_CHUNKS // IDX_BLK, C1_CHUNKS // IDX_BLK)

    def block(b, carry):
        off = start_chunk + b * IDX_BLK
        pltpu.sync_copy(src_hbm.at[pl.ds(off, IDX_BLK)], src_v)
        pltpu.sync_copy(dst_hbm.at[pl.ds(off, IDX_BLK)], dst_v)
        pltpu.async_copy(hp_hbm.at[src_v.at[0]], buf0, sem0)

        def body(t, c2):
            j0 = 2 * t
            j1 = j0 + 1
            pltpu.async_copy(hp_hbm.at[src_v.at[j1]], buf1, sem1)
            pltpu.make_async_copy(hp_hbm.at[src_v.at[j0]], buf0, sem0).wait()
            pltpu.sync_copy(buf0, acc.at[dst_v.at[j0]], add=True)
            jn = jnp.minimum(j0 + 2, IDX_BLK - 2)
            pltpu.async_copy(hp_hbm.at[src_v.at[jn]], buf0, sem0)
            pltpu.make_async_copy(hp_hbm.at[src_v.at[j1]], buf1, sem1).wait()
            pltpu.sync_copy(buf1, acc.at[dst_v.at[j1]], add=True)
            return c2
        lax.fori_loop(0, IDX_BLK // 2, body, 0)
        pltpu.make_async_copy(hp_hbm.at[src_v.at[IDX_BLK - 2]], buf0, sem0).wait()
        return carry
    lax.fori_loop(0, nblocks, block, 0)

    plsc.subcore_barrier()
    for k in range(WB_CHUNKS):
        pltpu.sync_copy(acc.at[pl.ds(base + k * CHUNK, CHUNK)],
                        out_hbm.at[c, pl.ds(base + k * CHUNK, CHUNK)])


BN = 1000


def _dinv_of(deg_ref):
    d = deg_ref[0] + deg_ref[1]
    return lax.rsqrt(1.0 + d[:, 0:1])


def _mm1_body(deg_ref, x_ref, w_ref, hp_ref):
    dinv = _dinv_of(deg_ref)
    h = jnp.dot(x_ref[...], w_ref[...], preferred_element_type=jnp.float32)
    hp_ref[...] = dinv * h


def _mid_body(deg_ref, agg_ref, hp1_ref, b1_ref, w2_ref, hp2_ref):
    dinv = _dinv_of(deg_ref)
    aggsum = agg_ref[0] + agg_ref[1]
    z = jnp.maximum(dinv * (aggsum + hp1_ref[...]) + b1_ref[...], 0.0)
    hp2_ref[...] = dinv * jnp.dot(z, w2_ref[...], preferred_element_type=jnp.float32)


def _out_body(deg_ref, agg_ref, hp2_ref, b2_ref, o_ref):
    dinv = _dinv_of(deg_ref)
    aggsum = agg_ref[0] + agg_ref[1]
    o_ref[...] = dinv * (aggsum + hp2_ref[...]) + b2_ref[...]


_deg_spec = pl.BlockSpec((NC, BN, LANES), lambda i: (0, i, 0))
_agg_spec = pl.BlockSpec((NC, BN, D), lambda i: (0, i, 0))
_row_spec = pl.BlockSpec((BN, D), lambda i: (i, 0))
_w_spec = pl.BlockSpec((D, D), lambda i: (0, 0))
_b_spec = pl.BlockSpec((1, D), lambda i: (0, 0))

_mm1 = pl.pallas_call(
    _mm1_body,
    grid=(N // BN,),
    in_specs=[_deg_spec, _row_spec, _w_spec],
    out_specs=_row_spec,
    out_shape=jax.ShapeDtypeStruct((N, D), jnp.float32),
)

_mid = pl.pallas_call(
    _mid_body,
    grid=(N // BN,),
    in_specs=[_deg_spec, _agg_spec, _row_spec, _b_spec, _w_spec],
    out_specs=_row_spec,
    out_shape=jax.ShapeDtypeStruct((N, D), jnp.float32),
)

_outk = pl.pallas_call(
    _out_body,
    grid=(N // BN,),
    in_specs=[_deg_spec, _agg_spec, _row_spec, _b_spec],
    out_specs=_row_spec,
    out_shape=jax.ShapeDtypeStruct((N, D), jnp.float32),
)


def kernel(x, edge_index, W1, b1, W2, b2):
    e = edge_index.shape[1]
    pad = E_PAD - e
    src = jnp.concatenate([edge_index[0], jnp.zeros((pad,), jnp.int32)])
    dst = jnp.concatenate([edge_index[1], jnp.full((pad,), DUMMY_DST, jnp.int32)])
    src_p = src.reshape(TOT_CHUNKS, CHUNK)
    dst_p = dst.reshape(TOT_CHUNKS, CHUNK)

    deg_parts = _deg_kernel(dst_p)
    hp1 = _mm1(deg_parts, x, W1)
    agg1 = _edge_kernel(src_p, dst_p, hp1)
    hp2 = _mid(deg_parts, agg1, hp1, b1.reshape(1, D), W2)
    agg2 = _edge_kernel(src_p, dst_p, hp2)
    return _outk(deg_parts, agg2, hp2, b2.reshape(1, D))
